# baseline (speedup 1.0000x reference)
import functools

import jax
import jax.numpy as jnp
import numpy as np
from jax import lax
from jax.experimental import pallas as pl
from jax.experimental.pallas import tpu as pltpu
from jax.experimental.pallas import tpu_sc as plsc

F0, F1, F2 = 5, 6, 2
EMB = 128
E = 320000
NROWS = F0 * F1 * F2

NC, NS = 2, 16
NW = NC * NS
PER_W = E // NW
CHUNK = 80
NCH = PER_W // CHUNK
NBUF = 5

IDX_ROWS = 20
IDX_GRID = E // (IDX_ROWS * 128)

_M_NP = np.zeros((3 * 128, 128), np.float32)
for _j in range(3 * 128):
    _M_NP[_j, _j // 3] = (F1 * F2, F2, 1)[_j % 3]


def _idx_body(x_ref, m_ref, o_ref):
    x = x_ref[0].astype(jnp.float32)
    r = jnp.dot(x, m_ref[...], preferred_element_type=jnp.float32)
    o_ref[0] = r.astype(jnp.int32)


def _combined_idx(ea):
    xr = ea.reshape(IDX_GRID, IDX_ROWS, 3 * 128)
    out = pl.pallas_call(
        _idx_body,
        grid=(IDX_GRID,),
        in_specs=[
            pl.BlockSpec((1, IDX_ROWS, 3 * 128), lambda g: (g, 0, 0)),
            pl.BlockSpec((3 * 128, 128), lambda g: (0, 0)),
        ],
        out_specs=pl.BlockSpec((1, IDX_ROWS, 128), lambda g: (g, 0, 0)),
        out_shape=jax.ShapeDtypeStruct((IDX_GRID, IDX_ROWS, 128), jnp.int32),
    )(xr, jnp.asarray(_M_NP))
    return out.reshape(E)


def _sc_body(idx_hbm, w0_hbm, w1_hbm, w2_hbm, out_hbm,
             idx_v, rows_v, w0_v, w1_v, w2_v, c_loc, c_sh,
             isem, gsem, osem):
    sid = lax.axis_index("s")
    wid = sid * NC + lax.axis_index("c")
    wbase = wid * PER_W

    @pl.when(sid < 15)
    def _():
        pltpu.sync_copy(w0_hbm, w0_v)
        pltpu.sync_copy(w1_hbm, w1_v)
        pltpu.sync_copy(w2_hbm, w2_v)
        for q in range(4):
            r = sid * 4 + q
            i0 = r // (F1 * F2)
            i1 = (r // F2) % F1
            i2 = r % F2
            for j in range(EMB // 16):
                c_loc[q, pl.ds(j * 16, 16)] = (
                    w0_v[pl.ds(i0 * EMB + j * 16, 16)]
                    + w1_v[pl.ds(i1 * EMB + j * 16, 16)]
                    + w2_v[pl.ds(i2 * EMB + j * 16, 16)]
                )
        pltpu.sync_copy(c_loc, c_sh.at[pl.ds(sid * 4, 4)])
    plsc.subcore_barrier()

    def fire_in(k, b):
        pltpu.async_copy(idx_hbm.at[pl.ds(wbase + k * CHUNK, CHUNK)],
                         idx_v.at[b], isem.at[b])

    def wait_in(k, b):
        pltpu.make_async_copy(idx_hbm.at[pl.ds(wbase + k * CHUNK, CHUNK)],
                              idx_v.at[b], isem.at[b]).wait()

    def fire_gather(k, b):
        pltpu.async_copy(c_sh.at[idx_v.at[b]], rows_v.at[b], gsem.at[b])

    def wait_gather(k, b):
        pltpu.make_async_copy(c_sh.at[idx_v.at[b]], rows_v.at[b],
                              gsem.at[b]).wait()

    def fire_out(k, b):
        pltpu.async_copy(rows_v.at[b], out_hbm.at[pl.ds(wbase + k * CHUNK, CHUNK)],
                         osem.at[b])

    def wait_out(k, b):
        pltpu.make_async_copy(rows_v.at[b], out_hbm.at[pl.ds(wbase + k * CHUNK, CHUNK)],
                              osem.at[b]).wait()

    def step(k, b, first_round, fire_next, has_prev=True):
        wait_in(k, b)
        if not first_round:
            wait_out(k - NBUF, b)
        fire_gather(k, b)
        if has_prev:
            bp = (b - 1) % NBUF
            wait_gather(k - 1, bp)
            fire_out(k - 1, bp)
            if fire_next:
                fire_in(k - 1 + NBUF, bp)

    for b in range(NBUF):
        fire_in(b, b)
    for b in range(NBUF):
        step(b, b, first_round=True, fire_next=True, has_prev=(b > 0))

    def super_step(g, carry):
        for b in range(NBUF):
            step(g * NBUF + b, b, first_round=False, fire_next=True)
        return carry

    lax.fori_loop(1, NCH // NBUF - 1, super_step, 0)
    for b in range(NBUF):
        k = (NCH - NBUF) + b
        step(k, b, first_round=False, fire_next=(k - 1 + NBUF < NCH))
    wait_gather(NCH - 1, (NCH - 1) % NBUF)
    fire_out(NCH - 1, (NCH - 1) % NBUF)
    for b in range(NBUF):
        wait_out((NCH - NBUF) + b, b)


@functools.partial(jax.jit, static_argnames=())
def _sc_lookup(idx, w0f, w1f, w2f):
    mesh = plsc.VectorSubcoreMesh(core_axis_name="c", subcore_axis_name="s")
    fn = pl.kernel(
        _sc_body,
        out_type=jax.ShapeDtypeStruct((E, EMB), jnp.float32),
        mesh=mesh,
        scratch_types=[
            pltpu.VMEM((NBUF, CHUNK), jnp.int32),
            pltpu.VMEM((NBUF, CHUNK, EMB), jnp.float32),
            pltpu.VMEM((F0 * EMB,), jnp.float32),
            pltpu.VMEM((F1 * EMB,), jnp.float32),
            pltpu.VMEM((F2 * EMB,), jnp.float32),
            pltpu.VMEM((4, EMB), jnp.float32),
            pltpu.VMEM_SHARED((NROWS, EMB), jnp.float32),
            pltpu.SemaphoreType.DMA((NBUF,)),
            pltpu.SemaphoreType.DMA((NBUF,)),
            pltpu.SemaphoreType.DMA((NBUF,)),
        ],
    )
    return fn(idx, w0f, w1f, w2f)


def kernel(edge_attr, W0, W1, W2):
    ea = jnp.asarray(edge_attr, jnp.int32)
    idx = _combined_idx(ea)
    return _sc_lookup(idx, W0.reshape(-1), W1.reshape(-1), W2.reshape(-1))


# speedup vs baseline: 3.1328x; 3.1328x over previous
import functools

import jax
import jax.numpy as jnp
import numpy as np
from jax import lax
from jax.experimental import pallas as pl
from jax.experimental.pallas import tpu as pltpu
from jax.experimental.pallas import tpu_sc as plsc

F0, F1, F2 = 5, 6, 2
EMB = 128
E = 320000
NROWS = F0 * F1 * F2

NC, NS = 2, 16
NW = NC * NS
PER_W = E // NW
CHUNK = 80
NCH = PER_W // CHUNK
NBUF = 5

def _sc_body(a0_hbm, a1_hbm, a2_hbm, w0_hbm, w1_hbm, w2_hbm, out_hbm,
             ab_v, idx_v, rows_v, w0_v, w1_v, w2_v, c_loc, c_sh,
             isem, gsem, osem):
    sid = lax.axis_index("s")
    wid = sid * NC + lax.axis_index("c")
    wbase = wid * PER_W

    @pl.when(sid < 15)
    def _():
        pltpu.sync_copy(w0_hbm, w0_v)
        pltpu.sync_copy(w1_hbm, w1_v)
        pltpu.sync_copy(w2_hbm, w2_v)
        for q in range(4):
            r = sid * 4 + q
            i0 = r // (F1 * F2)
            i1 = (r // F2) % F1
            i2 = r % F2
            for j in range(EMB // 16):
                c_loc[q, pl.ds(j * 16, 16)] = (
                    w0_v[pl.ds(i0 * EMB + j * 16, 16)]
                    + w1_v[pl.ds(i1 * EMB + j * 16, 16)]
                    + w2_v[pl.ds(i2 * EMB + j * 16, 16)]
                )
        pltpu.sync_copy(c_loc, c_sh.at[pl.ds(sid * 4, 4)])
    plsc.subcore_barrier()

    def fire_in(k, b):
        s = pl.ds(wbase + k * CHUNK, CHUNK)
        pltpu.async_copy(a0_hbm.at[s], ab_v.at[3 * b], isem.at[b])
        pltpu.async_copy(a1_hbm.at[s], ab_v.at[3 * b + 1], isem.at[b])
        pltpu.async_copy(a2_hbm.at[s], ab_v.at[3 * b + 2], isem.at[b])

    def wait_in(k, b):
        s = pl.ds(wbase + k * CHUNK, CHUNK)
        pltpu.make_async_copy(a0_hbm.at[s], ab_v.at[3 * b], isem.at[b]).wait()
        pltpu.make_async_copy(a1_hbm.at[s], ab_v.at[3 * b + 1], isem.at[b]).wait()
        pltpu.make_async_copy(a2_hbm.at[s], ab_v.at[3 * b + 2], isem.at[b]).wait()

    def combine(b):
        for i in range(CHUNK // 16):
            idx_v[b, pl.ds(i * 16, 16)] = (
                ab_v[3 * b, pl.ds(i * 16, 16)] * (F1 * F2)
                + ab_v[3 * b + 1, pl.ds(i * 16, 16)] * F2
                + ab_v[3 * b + 2, pl.ds(i * 16, 16)]
            )

    def fire_gather(k, b):
        pltpu.async_copy(c_sh.at[idx_v.at[b]], rows_v.at[b], gsem.at[b])

    def wait_gather(k, b):
        pltpu.make_async_copy(c_sh.at[idx_v.at[b]], rows_v.at[b],
                              gsem.at[b]).wait()

    def fire_out(k, b):
        pltpu.async_copy(rows_v.at[b], out_hbm.at[pl.ds(wbase + k * CHUNK, CHUNK)],
                         osem.at[b])

    def wait_out(k, b):
        pltpu.make_async_copy(rows_v.at[b], out_hbm.at[pl.ds(wbase + k * CHUNK, CHUNK)],
                              osem.at[b]).wait()

    def step(k, b, first_round, fire_next, has_prev=True):
        wait_in(k, b)
        combine(b)
        if not first_round:
            wait_out(k - NBUF, b)
        fire_gather(k, b)
        if has_prev:
            bp = (b - 1) % NBUF
            wait_gather(k - 1, bp)
            fire_out(k - 1, bp)
            if fire_next:
                fire_in(k - 1 + NBUF, bp)

    for b in range(NBUF):
        fire_in(b, b)
    for b in range(NBUF):
        step(b, b, first_round=True, fire_next=True, has_prev=(b > 0))

    def super_step(g, carry):
        for b in range(NBUF):
            step(g * NBUF + b, b, first_round=False, fire_next=True)
        return carry

    lax.fori_loop(1, NCH // NBUF - 1, super_step, 0)
    for b in range(NBUF):
        k = (NCH - NBUF) + b
        step(k, b, first_round=False, fire_next=(k - 1 + NBUF < NCH))
    wait_gather(NCH - 1, (NCH - 1) % NBUF)
    fire_out(NCH - 1, (NCH - 1) % NBUF)
    for b in range(NBUF):
        wait_out((NCH - NBUF) + b, b)


@functools.partial(jax.jit, static_argnames=())
def _sc_lookup(a0, a1, a2, w0f, w1f, w2f):
    mesh = plsc.VectorSubcoreMesh(core_axis_name="c", subcore_axis_name="s")
    fn = pl.kernel(
        _sc_body,
        out_type=jax.ShapeDtypeStruct((E, EMB), jnp.float32),
        mesh=mesh,
        scratch_types=[
            pltpu.VMEM((NBUF * 3, CHUNK), jnp.int32),
            pltpu.VMEM((NBUF, CHUNK), jnp.int32),
            pltpu.VMEM((NBUF, CHUNK, EMB), jnp.float32),
            pltpu.VMEM((F0 * EMB,), jnp.float32),
            pltpu.VMEM((F1 * EMB,), jnp.float32),
            pltpu.VMEM((F2 * EMB,), jnp.float32),
            pltpu.VMEM((4, EMB), jnp.float32),
            pltpu.VMEM_SHARED((NROWS, EMB), jnp.float32),
            pltpu.SemaphoreType.DMA((NBUF,)),
            pltpu.SemaphoreType.DMA((NBUF,)),
            pltpu.SemaphoreType.DMA((NBUF,)),
        ],
    )
    return fn(a0, a1, a2, w0f, w1f, w2f)


def kernel(edge_attr, W0, W1, W2):
    ea = jnp.asarray(edge_attr, jnp.int32)
    return _sc_lookup(ea[:, 0], ea[:, 1], ea[:, 2],
                      W0.reshape(-1), W1.reshape(-1), W2.reshape(-1))


# speedup vs baseline: 3.1552x; 1.0071x over previous
import functools

import jax
import jax.numpy as jnp
import numpy as np
from jax import lax
from jax.experimental import pallas as pl
from jax.experimental.pallas import tpu as pltpu
from jax.experimental.pallas import tpu_sc as plsc

F0, F1, F2 = 5, 6, 2
EMB = 128
E = 320000
NROWS = F0 * F1 * F2

NC, NS = 2, 16
NW = NC * NS
PER_W = E // NW
CHUNK = 80
NCH = PER_W // CHUNK
NBUF = 5

def _sc_body(a0_hbm, a1_hbm, a2_hbm, w0_hbm, w1_hbm, w2_hbm, out_hbm,
             ab_v, idx_v, rows_v, w0_v, w1_v, w2_v, c_loc, c_sh,
             isem, gsem, osem):
    sid = lax.axis_index("s")
    wid = sid * NC + lax.axis_index("c")
    wbase = wid * PER_W

    @pl.when(sid < 15)
    def _():
        pltpu.sync_copy(w0_hbm, w0_v)
        pltpu.sync_copy(w1_hbm, w1_v)
        pltpu.sync_copy(w2_hbm, w2_v)
        for q in range(4):
            r = sid * 4 + q
            i0 = r // (F1 * F2)
            i1 = (r // F2) % F1
            i2 = r % F2
            for j in range(EMB // 16):
                c_loc[q, pl.ds(j * 16, 16)] = (
                    w0_v[pl.ds(i0 * EMB + j * 16, 16)]
                    + w1_v[pl.ds(i1 * EMB + j * 16, 16)]
                    + w2_v[pl.ds(i2 * EMB + j * 16, 16)]
                )
        pltpu.sync_copy(c_loc, c_sh.at[pl.ds(sid * 4, 4)])
    plsc.subcore_barrier()

    def fire_in(k, b):
        s = pl.ds(wbase + k * CHUNK, CHUNK)
        pltpu.async_copy(a0_hbm.at[s], ab_v.at[3 * b], isem.at[b])
        pltpu.async_copy(a1_hbm.at[s], ab_v.at[3 * b + 1], isem.at[b])
        pltpu.async_copy(a2_hbm.at[s], ab_v.at[3 * b + 2], isem.at[b])

    def wait_in(k, b):
        s = pl.ds(wbase + k * CHUNK, CHUNK)
        pltpu.make_async_copy(a0_hbm.at[s], ab_v.at[3 * b], isem.at[b]).wait()
        pltpu.make_async_copy(a1_hbm.at[s], ab_v.at[3 * b + 1], isem.at[b]).wait()
        pltpu.make_async_copy(a2_hbm.at[s], ab_v.at[3 * b + 2], isem.at[b]).wait()

    def combine(b):
        for i in range(CHUNK // 16):
            idx_v[b, pl.ds(i * 16, 16)] = (
                ab_v[3 * b, pl.ds(i * 16, 16)] * (F1 * F2)
                + ab_v[3 * b + 1, pl.ds(i * 16, 16)] * F2
                + ab_v[3 * b + 2, pl.ds(i * 16, 16)]
            )

    def fire_gather(k, b):
        pltpu.async_copy(c_sh.at[idx_v.at[b]], rows_v.at[b], gsem.at[b])

    def wait_gather(k, b):
        pltpu.make_async_copy(c_sh.at[idx_v.at[b]], rows_v.at[b],
                              gsem.at[b]).wait()

    def fire_out(k, b):
        pltpu.async_copy(rows_v.at[b], out_hbm.at[pl.ds(wbase + k * CHUNK, CHUNK)],
                         osem.at[b])

    def wait_out(k, b):
        pltpu.make_async_copy(rows_v.at[b], out_hbm.at[pl.ds(wbase + k * CHUNK, CHUNK)],
                              osem.at[b]).wait()

    GLAG = 2

    def step(k, b, first_round, fire_next, has_prev=True):
        wait_in(k, b)
        combine(b)
        if not first_round:
            wait_out(k - NBUF, b)
        fire_gather(k, b)
        if has_prev:
            bp = (b - GLAG) % NBUF
            wait_gather(k - GLAG, bp)
            fire_out(k - GLAG, bp)
            if fire_next:
                fire_in(k - GLAG + NBUF, bp)

    for b in range(NBUF):
        fire_in(b, b)
    for b in range(NBUF):
        step(b, b, first_round=True, fire_next=True, has_prev=(b >= GLAG))

    def super_step(g, carry):
        for b in range(NBUF):
            step(g * NBUF + b, b, first_round=False, fire_next=True)
        return carry

    lax.fori_loop(1, NCH // NBUF - 1, super_step, 0)
    for b in range(NBUF):
        k = (NCH - NBUF) + b
        step(k, b, first_round=False, fire_next=(k - GLAG + NBUF < NCH))
    for k in range(NCH - GLAG, NCH):
        wait_gather(k, k % NBUF)
        fire_out(k, k % NBUF)
    for b in range(NBUF):
        wait_out((NCH - NBUF) + b, b)


@functools.partial(jax.jit, static_argnames=())
def _sc_lookup(a0, a1, a2, w0f, w1f, w2f):
    mesh = plsc.VectorSubcoreMesh(core_axis_name="c", subcore_axis_name="s")
    fn = pl.kernel(
        _sc_body,
        out_type=jax.ShapeDtypeStruct((E, EMB), jnp.float32),
        mesh=mesh,
        scratch_types=[
            pltpu.VMEM((NBUF * 3, CHUNK), jnp.int32),
            pltpu.VMEM((NBUF, CHUNK), jnp.int32),
            pltpu.VMEM((NBUF, CHUNK, EMB), jnp.float32),
            pltpu.VMEM((F0 * EMB,), jnp.float32),
            pltpu.VMEM((F1 * EMB,), jnp.float32),
            pltpu.VMEM((F2 * EMB,), jnp.float32),
            pltpu.VMEM((4, EMB), jnp.float32),
            pltpu.VMEM_SHARED((NROWS, EMB), jnp.float32),
            pltpu.SemaphoreType.DMA((NBUF,)),
            pltpu.SemaphoreType.DMA((NBUF,)),
            pltpu.SemaphoreType.DMA((NBUF,)),
        ],
    )
    return fn(a0, a1, a2, w0f, w1f, w2f)


def kernel(edge_attr, W0, W1, W2):
    ea = jnp.asarray(edge_attr, jnp.int32)
    return _sc_lookup(ea[:, 0], ea[:, 1], ea[:, 2],
                      W0.reshape(-1), W1.reshape(-1), W2.reshape(-1))


# speedup vs baseline: 3.1585x; 1.0010x over previous
import functools

import jax
import jax.numpy as jnp
from jax import lax
from jax.experimental import pallas as pl
from jax.experimental.pallas import tpu as pltpu
from jax.experimental.pallas import tpu_sc as plsc

F0, F1, F2 = 5, 6, 2
EMB = 128
E = 320000
NROWS = F0 * F1 * F2

NC, NS = 2, 16
NW = NC * NS
PER_W = E // NW
CHUNK = 80
NCH = PER_W // CHUNK
NBUF = 5

def _sc_body(a0_hbm, a1_hbm, a2_hbm, w0_hbm, w1_hbm, w2_hbm, out_hbm,
             ab_v, idx_v, rows_v, w0_v, w1_v, w2_v, c_loc, c_sh,
             isem, gsem, osem):
    sid = lax.axis_index("s")
    wid = sid * NC + lax.axis_index("c")
    wbase = wid * PER_W

    @pl.when(sid < 15)
    def _():
        pltpu.sync_copy(w0_hbm, w0_v)
        pltpu.sync_copy(w1_hbm, w1_v)
        pltpu.sync_copy(w2_hbm, w2_v)
        for q in range(4):
            r = sid * 4 + q
            i0 = r // (F1 * F2)
            i1 = (r // F2) % F1
            i2 = r % F2
            for j in range(EMB // 16):
                c_loc[q, pl.ds(j * 16, 16)] = (
                    w0_v[pl.ds(i0 * EMB + j * 16, 16)]
                    + w1_v[pl.ds(i1 * EMB + j * 16, 16)]
                    + w2_v[pl.ds(i2 * EMB + j * 16, 16)]
                )
        pltpu.sync_copy(c_loc, c_sh.at[pl.ds(sid * 4, 4)])
    plsc.subcore_barrier()

    def fire_in(k, b):
        s = pl.ds(wbase + k * CHUNK, CHUNK)
        pltpu.async_copy(a0_hbm.at[s], ab_v.at[3 * b], isem.at[b])
        pltpu.async_copy(a1_hbm.at[s], ab_v.at[3 * b + 1], isem.at[b])
        pltpu.async_copy(a2_hbm.at[s], ab_v.at[3 * b + 2], isem.at[b])

    def wait_in(k, b):
        s = pl.ds(wbase + k * CHUNK, CHUNK)
        pltpu.make_async_copy(a0_hbm.at[s], ab_v.at[3 * b], isem.at[b]).wait()
        pltpu.make_async_copy(a1_hbm.at[s], ab_v.at[3 * b + 1], isem.at[b]).wait()
        pltpu.make_async_copy(a2_hbm.at[s], ab_v.at[3 * b + 2], isem.at[b]).wait()

    def combine(b):
        for i in range(CHUNK // 16):
            idx_v[b, pl.ds(i * 16, 16)] = (
                ab_v[3 * b, pl.ds(i * 16, 16)] * (F1 * F2)
                + ab_v[3 * b + 1, pl.ds(i * 16, 16)] * F2
                + ab_v[3 * b + 2, pl.ds(i * 16, 16)]
            )

    def fire_gather(k, b):
        pltpu.async_copy(c_sh.at[idx_v.at[b]], rows_v.at[b], gsem.at[b])

    def wait_gather(k, b):
        pltpu.make_async_copy(c_sh.at[idx_v.at[b]], rows_v.at[b],
                              gsem.at[b]).wait()

    def fire_out(k, b):
        pltpu.async_copy(rows_v.at[b], out_hbm.at[pl.ds(wbase + k * CHUNK, CHUNK)],
                         osem.at[b])

    def wait_out(k, b):
        pltpu.make_async_copy(rows_v.at[b], out_hbm.at[pl.ds(wbase + k * CHUNK, CHUNK)],
                              osem.at[b]).wait()

    GLAG = 2

    def step(k, b, first_round, fire_next, has_prev=True):
        wait_in(k, b)
        combine(b)
        if not first_round:
            wait_out(k - NBUF, b)
        fire_gather(k, b)
        if has_prev:
            bp = (b - GLAG) % NBUF
            wait_gather(k - GLAG, bp)
            fire_out(k - GLAG, bp)
            if fire_next:
                fire_in(k - GLAG + NBUF, bp)

    for b in range(NBUF):
        fire_in(b, b)
    for b in range(NBUF):
        step(b, b, first_round=True, fire_next=True, has_prev=(b >= GLAG))

    def super_step(g, carry):
        for b in range(NBUF):
            step(g * NBUF + b, b, first_round=False, fire_next=True)
        return carry

    lax.fori_loop(1, NCH // NBUF - 1, super_step, 0)
    for b in range(NBUF):
        k = (NCH - NBUF) + b
        step(k, b, first_round=False, fire_next=(k - GLAG + NBUF < NCH))
    for k in range(NCH - GLAG, NCH):
        wait_gather(k, k % NBUF)
        fire_out(k, k % NBUF)
    for b in range(NBUF):
        wait_out((NCH - NBUF) + b, b)


@functools.partial(jax.jit, static_argnames=())
def _sc_lookup(a0, a1, a2, w0f, w1f, w2f):
    mesh = plsc.VectorSubcoreMesh(core_axis_name="c", subcore_axis_name="s")
    fn = pl.kernel(
        _sc_body,
        out_type=jax.ShapeDtypeStruct((E, EMB), jnp.float32),
        mesh=mesh,
        scratch_types=[
            pltpu.VMEM((NBUF * 3, CHUNK), jnp.int32),
            pltpu.VMEM((NBUF, CHUNK), jnp.int32),
            pltpu.VMEM((NBUF, CHUNK, EMB), jnp.float32),
            pltpu.VMEM((F0 * EMB,), jnp.float32),
            pltpu.VMEM((F1 * EMB,), jnp.float32),
            pltpu.VMEM((F2 * EMB,), jnp.float32),
            pltpu.VMEM((4, EMB), jnp.float32),
            pltpu.VMEM_SHARED((NROWS, EMB), jnp.float32),
            pltpu.SemaphoreType.DMA((NBUF,)),
            pltpu.SemaphoreType.DMA((NBUF,)),
            pltpu.SemaphoreType.DMA((NBUF,)),
        ],
    )
    return fn(a0, a1, a2, w0f, w1f, w2f)


def kernel(edge_attr, W0, W1, W2):
    ea = jnp.asarray(edge_attr, jnp.int32)
    return _sc_lookup(ea[:, 0], ea[:, 1], ea[:, 2],
                      W0.reshape(-1), W1.reshape(-1), W2.reshape(-1))
